# Initial kernel scaffold; baseline (speedup 1.0000x reference)
import jax
import jax.numpy as jnp
from jax.experimental import pallas as pl


def kernel(inputs, graph, weight, bias):
    raise NotImplementedError("write your pallas kernel here")


# speedup vs baseline: 3.5051x; 3.5051x over previous
import functools

import jax
import jax.numpy as jnp
from jax.experimental import pallas as pl
from jax.experimental.pallas import tpu as pltpu


def _prep_kernel(g_ref, x_ref, d_ref, y_ref):
    s = jnp.sum(g_ref[...], axis=1, keepdims=True)
    d = jax.lax.rsqrt(s)
    d_ref[...] = d
    y_ref[...] = x_ref[...] * d


def _main_kernel(g_ref, y_ref, x_ref, d_ref, w_ref, b_ref, o_ref, *, batch, tc):
    z = jnp.dot(g_ref[...], y_ref[...], preferred_element_type=jnp.float32)
    zp = z * d_ref[...]
    x = x_ref[...]
    w = w_ref[...]
    bias = b_ref[...]
    for b in range(batch):
        sl = slice(b * tc, (b + 1) * tc)
        sb = jnp.concatenate([x[:, sl], zp[:, sl]], axis=1)
        ob = jnp.dot(sb, w, preferred_element_type=jnp.float32) + bias
        o_ref[b] = jnp.maximum(ob, 0.0)


def kernel(inputs, graph, weight, bias):
    B, N, T, C = inputs.shape
    D = weight.shape[-1]
    BTC = B * T * C
    TC = T * C
    TD = T * D

    x2 = inputs.transpose(1, 0, 2, 3).reshape(N, BTC)

    w0 = weight[0, 0]
    w1 = weight[1, 0]
    eye = jnp.eye(T, dtype=weight.dtype)
    wa = (eye[:, None, :, None] * (w0 + w1)[None, :, None, :]).reshape(TC, TD)
    wb = (eye[:, None, :, None] * (-w1)[None, :, None, :]).reshape(TC, TD)
    wbig = jnp.concatenate([wa, wb], axis=0)
    bias_t = jnp.tile(bias.reshape(1, D), (1, T))

    bn = 256
    d_arr, y = pl.pallas_call(
        _prep_kernel,
        grid=(N // bn,),
        in_specs=[
            pl.BlockSpec((bn, N), lambda i: (i, 0)),
            pl.BlockSpec((bn, BTC), lambda i: (i, 0)),
        ],
        out_specs=[
            pl.BlockSpec((bn, 1), lambda i: (i, 0)),
            pl.BlockSpec((bn, BTC), lambda i: (i, 0)),
        ],
        out_shape=[
            jax.ShapeDtypeStruct((N, 1), jnp.float32),
            jax.ShapeDtypeStruct((N, BTC), jnp.float32),
        ],
        compiler_params=pltpu.CompilerParams(
            dimension_semantics=("parallel",),
        ),
    )(graph, x2)

    out = pl.pallas_call(
        functools.partial(_main_kernel, batch=B, tc=TC),
        grid=(N // bn,),
        in_specs=[
            pl.BlockSpec((bn, N), lambda i: (i, 0)),
            pl.BlockSpec((N, BTC), lambda i: (0, 0)),
            pl.BlockSpec((bn, BTC), lambda i: (i, 0)),
            pl.BlockSpec((bn, 1), lambda i: (i, 0)),
            pl.BlockSpec((2 * TC, TD), lambda i: (0, 0)),
            pl.BlockSpec((1, TD), lambda i: (0, 0)),
        ],
        out_specs=pl.BlockSpec((B, bn, TD), lambda i: (0, i, 0)),
        out_shape=jax.ShapeDtypeStruct((B, N, TD), jnp.float32),
        compiler_params=pltpu.CompilerParams(
            dimension_semantics=("parallel",),
        ),
    )(graph, y, x2, d_arr, wbig, bias_t)

    return out.reshape(B, N, T, D)


# speedup vs baseline: 3.5155x; 1.0030x over previous
import functools

import jax
import jax.numpy as jnp
from jax.experimental import pallas as pl
from jax.experimental.pallas import tpu as pltpu


def _phased_kernel(g_ref, x_ref, w_ref, b_ref, o_ref, gs_ref, s_ref, y_ref,
                   *, nsteps, bn, batch, tc):
    i = pl.program_id(0)

    @pl.when(i < nsteps)
    def _phase1():
        g = g_ref[...]
        gs_ref[pl.ds(i * bn, bn), :] = g.astype(jnp.bfloat16)
        s_ref[pl.ds(i * bn, bn), :] = jnp.sum(g, axis=1, keepdims=True)

    @pl.when(i == nsteps)
    def _transition():
        d = jax.lax.rsqrt(s_ref[...])
        s_ref[...] = d
        y_ref[...] = x_ref[...] * d

    @pl.when(i >= nsteps)
    def _phase2():
        r = (i - nsteps) * bn
        z = jnp.dot(gs_ref[pl.ds(r, bn), :].astype(jnp.float32), y_ref[...],
                    preferred_element_type=jnp.float32)
        zp = z * s_ref[pl.ds(r, bn), :]
        x = x_ref[pl.ds(r, bn), :]
        w = w_ref[...]
        bias = b_ref[...]
        for b in range(batch):
            sl = slice(b * tc, (b + 1) * tc)
            sb = jnp.concatenate([x[:, sl], zp[:, sl]], axis=1)
            ob = jnp.dot(sb, w, preferred_element_type=jnp.float32) + bias
            o_ref[b] = jnp.maximum(ob, 0.0)


def kernel(inputs, graph, weight, bias):
    B, N, T, C = inputs.shape
    D = weight.shape[-1]
    BTC = B * T * C
    TC = T * C
    TD = T * D

    x2 = inputs.transpose(1, 0, 2, 3).reshape(N, BTC)

    w0 = weight[0, 0]
    w1 = weight[1, 0]
    eye = jnp.eye(T, dtype=weight.dtype)
    wa = (eye[:, None, :, None] * (w0 + w1)[None, :, None, :]).reshape(TC, TD)
    wb = (eye[:, None, :, None] * (-w1)[None, :, None, :]).reshape(TC, TD)
    wbig = jnp.concatenate([wa, wb], axis=0)
    bias_t = jnp.tile(bias.reshape(1, D), (1, T))

    bn = 128
    nsteps = N // bn

    out = pl.pallas_call(
        functools.partial(_phased_kernel, nsteps=nsteps, bn=bn, batch=B, tc=TC),
        grid=(2 * nsteps,),
        in_specs=[
            pl.BlockSpec((bn, N), lambda i: (jnp.minimum(i, nsteps - 1), 0)),
            pl.BlockSpec((N, BTC), lambda i: (0, 0)),
            pl.BlockSpec((2 * TC, TD), lambda i: (0, 0)),
            pl.BlockSpec((1, TD), lambda i: (0, 0)),
        ],
        out_specs=pl.BlockSpec(
            (B, bn, TD), lambda i: (0, jnp.maximum(i - nsteps, 0), 0)),
        out_shape=jax.ShapeDtypeStruct((B, N, TD), jnp.float32),
        scratch_shapes=[
            pltpu.VMEM((N, N), jnp.bfloat16),
            pltpu.VMEM((N, 1), jnp.float32),
            pltpu.VMEM((N, BTC), jnp.float32),
        ],
        compiler_params=pltpu.CompilerParams(
            dimension_semantics=("arbitrary",),
            vmem_limit_bytes=128 * 1024 * 1024,
        ),
    )(graph, x2, wbig, bias_t)

    return out.reshape(B, N, T, D)


# speedup vs baseline: 3.8262x; 1.0884x over previous
import functools

import jax
import jax.numpy as jnp
from jax.experimental import pallas as pl
from jax.experimental.pallas import tpu as pltpu

NBUF = 8
OBUF = 2
FBN = 64
BN = 128


def _spatial_conv_kernel(g_hbm, x_ref, w_ref, b_ref, o_hbm,
                         buf, gs_ref, s_ref, y_ref, ostg, isem, osem,
                         *, n, batch, tc):
    nblk = n // BN

    nfblk = n // FBN

    def fetch(idx):
        return pltpu.make_async_copy(
            g_hbm.at[pl.ds(idx * FBN, FBN), :], buf.at[idx % NBUF],
            isem.at[idx % NBUF])

    def store(idx):
        return pltpu.make_async_copy(
            ostg.at[idx % OBUF], o_hbm.at[:, pl.ds(idx * BN, BN), :],
            osem.at[idx % OBUF])

    for k in range(NBUF):
        fetch(k).start()

    def p1_body(i, carry):
        fetch(i).wait()
        g = buf[i % NBUF]
        s_ref[pl.ds(i * FBN, FBN), :] = jnp.sum(g, axis=1, keepdims=True)
        gs_ref[pl.ds(i * FBN, FBN), :] = g.astype(jnp.bfloat16)

        @pl.when(i + NBUF < nfblk)
        def _():
            fetch(i + NBUF).start()

        return carry

    jax.lax.fori_loop(0, nfblk, p1_body, 0)

    d_all = jax.lax.rsqrt(s_ref[...])
    s_ref[...] = d_all
    y_ref[...] = (x_ref[...] * d_all).astype(jnp.bfloat16)

    w = w_ref[...]
    bias = b_ref[...]
    y = y_ref[...]

    def p2_body(i, carry):
        @pl.when(i >= OBUF)
        def _():
            store(i - OBUF).wait()

        r = i * BN
        z = jnp.dot(gs_ref[pl.ds(r, BN), :], y,
                    preferred_element_type=jnp.float32)
        zp = z * s_ref[pl.ds(r, BN), :]
        x = x_ref[pl.ds(r, BN), :]
        slot = i % OBUF
        for b in range(batch):
            sl = slice(b * tc, (b + 1) * tc)
            sb = jnp.concatenate([x[:, sl], zp[:, sl]], axis=1)
            ob = jnp.dot(sb, w, preferred_element_type=jnp.float32) + bias
            ostg[slot, b] = jnp.maximum(ob, 0.0)
        store(i).start()
        return carry

    jax.lax.fori_loop(0, nblk, p2_body, 0)

    for k in range(OBUF):
        store(nblk - OBUF + k).wait()


def kernel(inputs, graph, weight, bias):
    B, N, T, C = inputs.shape
    D = weight.shape[-1]
    BTC = B * T * C
    TC = T * C
    TD = T * D

    x2 = inputs.transpose(1, 0, 2, 3).reshape(N, BTC)

    w0 = weight[0, 0]
    w1 = weight[1, 0]
    eye = jnp.eye(T, dtype=weight.dtype)
    wa = (eye[:, None, :, None] * (w0 + w1)[None, :, None, :]).reshape(TC, TD)
    wb = (eye[:, None, :, None] * (-w1)[None, :, None, :]).reshape(TC, TD)
    wbig = jnp.concatenate([wa, wb], axis=0)
    bias_t = jnp.tile(bias.reshape(1, D), (1, T))

    out = pl.pallas_call(
        functools.partial(_spatial_conv_kernel, n=N, batch=B, tc=TC),
        in_specs=[
            pl.BlockSpec(memory_space=pltpu.HBM),
            pl.BlockSpec(memory_space=pltpu.VMEM),
            pl.BlockSpec(memory_space=pltpu.VMEM),
            pl.BlockSpec(memory_space=pltpu.VMEM),
        ],
        out_specs=pl.BlockSpec(memory_space=pltpu.HBM),
        out_shape=jax.ShapeDtypeStruct((B, N, TD), jnp.float32),
        scratch_shapes=[
            pltpu.VMEM((NBUF, FBN, N), jnp.float32),
            pltpu.VMEM((N, N), jnp.bfloat16),
            pltpu.VMEM((N, 1), jnp.float32),
            pltpu.VMEM((N, BTC), jnp.bfloat16),
            pltpu.VMEM((OBUF, B, BN, TD), jnp.float32),
            pltpu.SemaphoreType.DMA((NBUF,)),
            pltpu.SemaphoreType.DMA((OBUF,)),
        ],
        compiler_params=pltpu.CompilerParams(
            vmem_limit_bytes=128 * 1024 * 1024,
        ),
    )(graph, x2, wbig, bias_t)

    return out.reshape(B, N, T, D)


# speedup vs baseline: 3.8393x; 1.0034x over previous
import functools

import jax
import jax.numpy as jnp
from jax.experimental import pallas as pl
from jax.experimental.pallas import tpu as pltpu

NBUF = 8
OBUF = 4
BN = 128


def _spatial_conv_kernel(g_hbm, x_ref, w_ref, b_ref, o_hbm,
                         buf, s_ref, y_ref, ostg, isem, osem,
                         *, n, batch, tc):
    nblk = n // BN

    def fetch(idx):
        return pltpu.make_async_copy(
            g_hbm.at[pl.ds((idx % nblk) * BN, BN), :], buf.at[idx % NBUF],
            isem.at[idx % NBUF])

    def store(idx):
        return pltpu.make_async_copy(
            ostg.at[idx % OBUF], o_hbm.at[:, pl.ds(idx * BN, BN), :],
            osem.at[idx % OBUF])

    for k in range(NBUF):
        fetch(k).start()

    def p1_body(i, carry):
        fetch(i).wait()
        s_ref[pl.ds(i * BN, BN), :] = jnp.sum(buf[i % NBUF], axis=1,
                                              keepdims=True)
        fetch(i + NBUF).start()
        return carry

    jax.lax.fori_loop(0, nblk, p1_body, 0)

    d_all = jax.lax.rsqrt(s_ref[...])
    s_ref[...] = d_all
    y_ref[...] = x_ref[...] * d_all

    w = w_ref[...]
    bias = b_ref[...]
    y = y_ref[...]

    def p2_body(i, carry):
        fetch(nblk + i).wait()

        @pl.when(i >= OBUF)
        def _():
            store(i - OBUF).wait()

        z = jnp.dot(buf[i % NBUF], y, preferred_element_type=jnp.float32)
        r = i * BN
        zp = z * s_ref[pl.ds(r, BN), :]
        x = x_ref[pl.ds(r, BN), :]
        slot = i % OBUF
        for b in range(batch):
            sl = slice(b * tc, (b + 1) * tc)
            sb = jnp.concatenate([x[:, sl], zp[:, sl]], axis=1)
            ob = jnp.dot(sb, w, preferred_element_type=jnp.float32) + bias
            ostg[slot, b] = jnp.maximum(ob, 0.0)
        store(i).start()

        @pl.when(nblk + i + NBUF < 2 * nblk)
        def _():
            fetch(nblk + i + NBUF).start()

        return carry

    jax.lax.fori_loop(0, nblk, p2_body, 0)

    for k in range(OBUF):
        store(nblk - OBUF + k).wait()


def kernel(inputs, graph, weight, bias):
    B, N, T, C = inputs.shape
    D = weight.shape[-1]
    BTC = B * T * C
    TC = T * C
    TD = T * D

    x2 = inputs.transpose(1, 0, 2, 3).reshape(N, BTC)

    w0 = weight[0, 0]
    w1 = weight[1, 0]
    eye = jnp.eye(T, dtype=weight.dtype)
    wa = (eye[:, None, :, None] * (w0 + w1)[None, :, None, :]).reshape(TC, TD)
    wb = (eye[:, None, :, None] * (-w1)[None, :, None, :]).reshape(TC, TD)
    wbig = jnp.concatenate([wa, wb], axis=0)
    bias_t = jnp.tile(bias.reshape(1, D), (1, T))

    out = pl.pallas_call(
        functools.partial(_spatial_conv_kernel, n=N, batch=B, tc=TC),
        in_specs=[
            pl.BlockSpec(memory_space=pltpu.HBM),
            pl.BlockSpec(memory_space=pltpu.VMEM),
            pl.BlockSpec(memory_space=pltpu.VMEM),
            pl.BlockSpec(memory_space=pltpu.VMEM),
        ],
        out_specs=pl.BlockSpec(memory_space=pltpu.HBM),
        out_shape=jax.ShapeDtypeStruct((B, N, TD), jnp.float32),
        scratch_shapes=[
            pltpu.VMEM((NBUF, BN, N), jnp.float32),
            pltpu.VMEM((N, 1), jnp.float32),
            pltpu.VMEM((N, BTC), jnp.float32),
            pltpu.VMEM((OBUF, B, BN, TD), jnp.float32),
            pltpu.SemaphoreType.DMA((NBUF,)),
            pltpu.SemaphoreType.DMA((OBUF,)),
        ],
        compiler_params=pltpu.CompilerParams(
            vmem_limit_bytes=128 * 1024 * 1024,
        ),
    )(graph, x2, wbig, bias_t)

    return out.reshape(B, N, T, D)
